# Initial kernel scaffold; baseline (speedup 1.0000x reference)
import jax
import jax.numpy as jnp
from jax.experimental import pallas as pl


def kernel(logit, target):
    raise NotImplementedError("write your pallas kernel here")


# speedup vs baseline: 2871.7501x; 2871.7501x over previous
import functools

import jax
import jax.numpy as jnp
from jax.experimental import pallas as pl
from jax.experimental.pallas import tpu as pltpu

_IGNORE = 255
_POS_MARGIN = 0.5
_NEG_MARGIN = 0.1


def _softplus(x):
    return jnp.maximum(x, 0.0) + jnp.log1p(jnp.exp(-jnp.abs(x)))


def _tile_kernel(logit_ref, tgt_ref, loss_ref, cnt_ref, *, c):
    x = logit_ref[...]
    tgt = tgt_ref[...]
    bw = x.shape[-1]

    mask = tgt != _IGNORE
    tgt0 = jnp.where(mask, tgt, 0)

    ji = jax.lax.broadcasted_iota(jnp.int32, (1, c, bw), 1)
    m1 = jnp.max(x, axis=1, keepdims=True)
    amax = jnp.min(jnp.where(x >= m1, ji, c), axis=1, keepdims=True)
    is_amax = ji == amax
    m2 = jnp.max(jnp.where(is_amax, -jnp.inf, x), axis=1, keepdims=True)
    sub = jnp.where(is_amax, m2, m1)
    score = x - sub

    is_tgt = ji == tgt0
    z = jnp.where(is_tgt, _POS_MARGIN - score, score + _NEG_MARGIN)
    w = jnp.where(is_tgt, jnp.float32(c), jnp.float32(1.0))
    maskf = mask.astype(jnp.float32)
    loss = jnp.sum(w * _softplus(z), axis=1, keepdims=True) * maskf

    @pl.when(pl.program_id(1) == 0)
    def _init():
        loss_ref[...] = loss
        cnt_ref[...] = maskf

    @pl.when(pl.program_id(1) != 0)
    def _acc():
        loss_ref[...] += loss
        cnt_ref[...] += maskf


def kernel(logit, target):
    n, c, h, w = logit.shape
    W = h * w
    bw = 4096
    while W % bw:
        bw //= 2
    k = W // bw

    logit3 = logit.reshape(n, c, W)
    target3 = target.reshape(n, 1, W)

    grid = (n, k)
    loss_p, cnt_p = pl.pallas_call(
        functools.partial(_tile_kernel, c=c),
        grid=grid,
        in_specs=[
            pl.BlockSpec((1, c, bw), lambda i, j: (i, 0, j)),
            pl.BlockSpec((1, 1, bw), lambda i, j: (i, 0, j)),
        ],
        out_specs=[
            pl.BlockSpec((1, 1, bw), lambda i, j: (i, 0, 0)),
            pl.BlockSpec((1, 1, bw), lambda i, j: (i, 0, 0)),
        ],
        out_shape=[
            jax.ShapeDtypeStruct((n, 1, bw), jnp.float32),
            jax.ShapeDtypeStruct((n, 1, bw), jnp.float32),
        ],
        compiler_params=pltpu.CompilerParams(
            dimension_semantics=("parallel", "arbitrary"),
        ),
    )(logit3, target3)

    return jnp.sum(loss_p) / (jnp.sum(cnt_p) * jnp.float32(c))
